# Initial kernel scaffold; baseline (speedup 1.0000x reference)
import jax
import jax.numpy as jnp
from jax.experimental import pallas as pl


def kernel(batch_confidence_maps, B, batch_warp_maks_list, record_len, warp_vis_list, warp_conf_list, warp_x_list, gauss_kernel):
    raise NotImplementedError("write your pallas kernel here")


# speedup vs baseline: 16.5530x; 16.5530x over previous
import functools

import ml_dtypes
import numpy as np
import jax
import jax.numpy as jnp
from jax.experimental import pallas as pl
from jax.experimental.pallas import tpu as pltpu

_K_RATIO = 0.5
_KSIZE = 5
_SIGMA = 1.0


def _gauss_2d_bf16():
    c = _KSIZE // 2
    x, y = np.mgrid[0 - c:_KSIZE - c, 0 - c:_KSIZE - c]
    gk = 1.0 / (2.0 * np.pi * _SIGMA) * np.exp(
        -(np.square(x) + np.square(y)) / (2.0 * np.square(_SIGMA)))
    gk32 = gk.astype(np.float32)
    return gk32.astype(ml_dtypes.bfloat16).astype(np.float32)


def _map_body(L, conf_ref, wm_ref, mask_ref, cnt_ref, pad_ref, tmp_ref):
    A = conf_ref.shape[2]
    H, W = mask_ref.shape[2], mask_ref.shape[3]
    K = int(H * W * _K_RATIO)
    P = _KSIZE // 2
    gw = _gauss_2d_bf16()

    m = conf_ref[0, 0, 0]
    for a in range(1, A):
        m = jnp.maximum(m, conf_ref[0, 0, a])
    s = jax.nn.sigmoid(m) * wm_ref[0, 0, 0]
    s = s.astype(jnp.bfloat16).astype(jnp.float32)

    pad_ref[...] = jnp.zeros_like(pad_ref)
    pad_ref[P:P + H, P:P + W] = s
    cm = jnp.zeros((H, W), jnp.float32)
    for dy in range(_KSIZE):
        tmp_ref[...] = pad_ref[dy:dy + H, :]
        for dx in range(_KSIZE):
            cm += jnp.float32(gw[dy, dx]) * tmp_ref[:, dx:dx + W]

    bits = jax.lax.bitcast_convert_type(cm, jnp.int32)

    def step(_, lohi):
        lo, hi = lohi
        mid = lo + (hi - lo + 1) // 2
        cnt = jnp.sum((bits >= mid).astype(jnp.int32))
        big = cnt >= K
        return jnp.where(big, mid, lo), jnp.where(big, hi, mid - 1)

    lo, _ = jax.lax.fori_loop(
        0, 30, step, (jnp.int32(0), jnp.int32(0x3F800000)))

    sel = (bits >= lo).astype(jnp.float32)
    cnt_ref[0, 0, 0] = jnp.sum(sel)
    is_ego = (pl.program_id(0) % L) == 0
    mask_ref[0, 0] = jnp.where(is_ego, jnp.float32(1.0), sel)


def kernel(batch_confidence_maps, B, batch_warp_maks_list, record_len,
           warp_vis_list, warp_conf_list, warp_x_list, gauss_kernel):
    Bs, L, A, H, W = batch_confidence_maps.shape
    BL = Bs * L
    P = _KSIZE // 2
    K = int(H * W * _K_RATIO)

    conf = batch_confidence_maps.reshape(BL, 1, A, H, W)
    wm = batch_warp_maks_list.reshape(BL, 1, 1, H, W)

    masks, counts = pl.pallas_call(
        functools.partial(_map_body, L),
        grid=(BL,),
        in_specs=[
            pl.BlockSpec((1, 1, A, H, W), lambda i: (i, 0, 0, 0, 0)),
            pl.BlockSpec((1, 1, 1, H, W), lambda i: (i, 0, 0, 0, 0)),
        ],
        out_specs=[
            pl.BlockSpec((1, 1, H, W), lambda i: (i, 0, 0, 0)),
            pl.BlockSpec((1, 1, 1), lambda i: (i, 0, 0),
                         memory_space=pltpu.SMEM),
        ],
        out_shape=[
            jax.ShapeDtypeStruct((BL, 1, H, W), jnp.float32),
            jax.ShapeDtypeStruct((BL, 1, 1), jnp.float32),
        ],
        scratch_shapes=[
            pltpu.VMEM((H + 2 * P, W + 2 * P), jnp.float32),
            pltpu.VMEM((H, W + 2 * P), jnp.float32),
        ],
        compiler_params=pltpu.CompilerParams(
            dimension_semantics=("arbitrary",)),
    )(conf, wm)

    counts = counts.reshape(Bs, L)
    rates = jnp.sum(counts[:, 1:], axis=1) / ((L - 1) * H * W)
    rate = jnp.sum(rates) / Bs
    return masks, rate


# speedup vs baseline: 37.3030x; 2.2535x over previous
import functools

import ml_dtypes
import numpy as np
import jax
import jax.numpy as jnp
from jax.experimental import pallas as pl
from jax.experimental.pallas import tpu as pltpu

_K_RATIO = 0.5
_KSIZE = 5
_SIGMA = 1.0


def _gauss_2d_bf16():
    c = _KSIZE // 2
    x, y = np.mgrid[0 - c:_KSIZE - c, 0 - c:_KSIZE - c]
    gk = 1.0 / (2.0 * np.pi * _SIGMA) * np.exp(
        -(np.square(x) + np.square(y)) / (2.0 * np.square(_SIGMA)))
    gk32 = gk.astype(np.float32)
    return gk32.astype(ml_dtypes.bfloat16).astype(np.float32)


def _batch_body(conf_ref, wm_ref, mask_ref, cnt_ref, pad_ref):
    L, A = conf_ref.shape[1], conf_ref.shape[2]
    H, W = mask_ref.shape[2], mask_ref.shape[3]
    K = int(H * W * _K_RATIO)
    P = _KSIZE // 2
    gw = _gauss_2d_bf16()

    m = conf_ref[0, :, 0]
    for a in range(1, A):
        m = jnp.maximum(m, conf_ref[0, :, a])
    s = jax.nn.sigmoid(m) * wm_ref[0, :, 0]
    s = s.astype(jnp.bfloat16).astype(jnp.float32)

    pad_ref[...] = jnp.zeros_like(pad_ref)
    pad_ref[:, P:P + H, P:P + W] = s
    cm = jnp.zeros((L, H, W), jnp.float32)
    for dy in range(_KSIZE):
        for dx in range(_KSIZE):
            cm += jnp.float32(gw[dy, dx]) * pad_ref[:, dy:dy + H, dx:dx + W]

    bits = jax.lax.bitcast_convert_type(cm, jnp.int32)

    def step(_, lohi):
        lo, hi = lohi
        mid = lo + (hi - lo + 1) // 2
        cnt = jnp.sum((bits >= mid).astype(jnp.int32), axis=(1, 2),
                      keepdims=True)
        big = cnt >= K
        return jnp.where(big, mid, lo), jnp.where(big, hi, mid - 1)

    lo0 = jnp.zeros((L, 1, 1), jnp.int32)
    hi0 = jnp.full((L, 1, 1), 0x3F800000, jnp.int32)
    lo, _ = jax.lax.fori_loop(0, 30, step, (lo0, hi0))

    sel = (bits >= lo).astype(jnp.float32)
    cnt = jnp.sum(sel, axis=(1, 2)).reshape(L, 1)
    cnt_ref[0] = jnp.broadcast_to(cnt, cnt_ref.shape[1:])
    is_ego = jax.lax.broadcasted_iota(jnp.int32, (L, 1, 1), 0) == 0
    mask_ref[0] = jnp.where(is_ego, jnp.float32(1.0), sel)


def kernel(batch_confidence_maps, B, batch_warp_maks_list, record_len,
           warp_vis_list, warp_conf_list, warp_x_list, gauss_kernel):
    Bs, L, A, H, W = batch_confidence_maps.shape
    P = _KSIZE // 2

    masks, counts = pl.pallas_call(
        _batch_body,
        grid=(Bs,),
        in_specs=[
            pl.BlockSpec((1, L, A, H, W), lambda b: (b, 0, 0, 0, 0)),
            pl.BlockSpec((1, L, 1, H, W), lambda b: (b, 0, 0, 0, 0)),
        ],
        out_specs=[
            pl.BlockSpec((1, L, H, W), lambda b: (b, 0, 0, 0)),
            pl.BlockSpec((1, L, 128), lambda b: (b, 0, 0)),
        ],
        out_shape=[
            jax.ShapeDtypeStruct((Bs, L, H, W), jnp.float32),
            jax.ShapeDtypeStruct((Bs, L, 128), jnp.float32),
        ],
        scratch_shapes=[
            pltpu.VMEM((L, H + 2 * P, W + 2 * P), jnp.float32),
        ],
        compiler_params=pltpu.CompilerParams(
            dimension_semantics=("arbitrary",)),
    )(batch_confidence_maps, batch_warp_maks_list)

    masks = masks.reshape(Bs * L, 1, H, W)

    counts = counts[:, :, 0]
    rates = jnp.sum(counts[:, 1:], axis=1) / ((L - 1) * H * W)
    rate = jnp.sum(rates) / Bs
    return masks, rate


# speedup vs baseline: 57.9036x; 1.5523x over previous
import functools

import ml_dtypes
import numpy as np
import jax
import jax.numpy as jnp
from jax.experimental import pallas as pl
from jax.experimental.pallas import tpu as pltpu

_K_RATIO = 0.5
_KSIZE = 5
_SIGMA = 1.0


def _gauss_2d_bf16():
    c = _KSIZE // 2
    x, y = np.mgrid[0 - c:_KSIZE - c, 0 - c:_KSIZE - c]
    gk = 1.0 / (2.0 * np.pi * _SIGMA) * np.exp(
        -(np.square(x) + np.square(y)) / (2.0 * np.square(_SIGMA)))
    gk32 = gk.astype(np.float32)
    return gk32.astype(ml_dtypes.bfloat16).astype(np.float32)


def _batch_body(bands_ref, conf_ref, wm_ref, mask_ref, cnt_ref, pad_ref,
                cm_ref):
    L, A = conf_ref.shape[1], conf_ref.shape[2]
    H, W = mask_ref.shape[2], mask_ref.shape[3]
    K = int(H * W * _K_RATIO)
    P = _KSIZE // 2

    m = conf_ref[0, :, 0]
    for a in range(1, A):
        m = jnp.maximum(m, conf_ref[0, :, a])
    s = jax.nn.sigmoid(m) * wm_ref[0, :, 0]

    pad_ref[...] = jnp.zeros_like(pad_ref)
    pad_ref[:, P:P + H, P:P + W] = s.astype(jnp.bfloat16)
    for l in range(L):
        acc = jnp.zeros((H, W), jnp.float32)
        for dy in range(_KSIZE):
            acc += jax.lax.dot_general(
                pad_ref[l, dy:dy + H, :], bands_ref[dy],
                (((1,), (0,)), ((), ())),
                preferred_element_type=jnp.float32)
        cm_ref[l] = acc

    bits = jax.lax.bitcast_convert_type(cm_ref[...], jnp.int32)

    def step(_, lohi):
        lo, hi = lohi
        mid = lo + (hi - lo + 1) // 2
        cnt = jnp.sum((bits >= mid).astype(jnp.int32), axis=(1, 2),
                      keepdims=True)
        big = cnt >= K
        return jnp.where(big, mid, lo), jnp.where(big, hi, mid - 1)

    lo0 = jnp.zeros((L, 1, 1), jnp.int32)
    hi0 = jnp.full((L, 1, 1), 0x3F800000, jnp.int32)
    lo, _ = jax.lax.fori_loop(0, 30, step, (lo0, hi0))

    sel = (bits >= lo).astype(jnp.float32)
    cnt = jnp.sum(sel, axis=(1, 2)).reshape(L, 1)
    cnt_ref[0] = jnp.broadcast_to(cnt, cnt_ref.shape[1:])
    is_ego = jax.lax.broadcasted_iota(jnp.int32, (L, 1, 1), 0) == 0
    mask_ref[0] = jnp.where(is_ego, jnp.float32(1.0), sel)


def kernel(batch_confidence_maps, B, batch_warp_maks_list, record_len,
           warp_vis_list, warp_conf_list, warp_x_list, gauss_kernel):
    Bs, L, A, H, W = batch_confidence_maps.shape
    P = _KSIZE // 2

    gw = _gauss_2d_bf16()
    bands_np = np.zeros((_KSIZE, W + 2 * P, W), np.float32)
    cols = np.arange(W)
    for dy in range(_KSIZE):
        for dx in range(_KSIZE):
            bands_np[dy, cols + dx, cols] = gw[dy, dx]
    bands = jnp.asarray(bands_np, dtype=jnp.bfloat16)

    masks, counts = pl.pallas_call(
        _batch_body,
        grid=(Bs,),
        in_specs=[
            pl.BlockSpec((_KSIZE, W + 2 * P, W), lambda b: (0, 0, 0)),
            pl.BlockSpec((1, L, A, H, W), lambda b: (b, 0, 0, 0, 0)),
            pl.BlockSpec((1, L, 1, H, W), lambda b: (b, 0, 0, 0, 0)),
        ],
        out_specs=[
            pl.BlockSpec((1, L, H, W), lambda b: (b, 0, 0, 0)),
            pl.BlockSpec((1, L, 128), lambda b: (b, 0, 0)),
        ],
        out_shape=[
            jax.ShapeDtypeStruct((Bs, L, H, W), jnp.float32),
            jax.ShapeDtypeStruct((Bs, L, 128), jnp.float32),
        ],
        scratch_shapes=[
            pltpu.VMEM((L, H + 2 * P, W + 2 * P), jnp.bfloat16),
            pltpu.VMEM((L, H, W), jnp.float32),
        ],
        compiler_params=pltpu.CompilerParams(
            dimension_semantics=("arbitrary",)),
    )(bands, batch_confidence_maps, batch_warp_maks_list)

    masks = masks.reshape(Bs * L, 1, H, W)

    counts = counts[:, :, 0]
    rates = jnp.sum(counts[:, 1:], axis=1) / ((L - 1) * H * W)
    rate = jnp.sum(rates) / Bs
    return masks, rate


# speedup vs baseline: 62.4112x; 1.0778x over previous
import functools

import ml_dtypes
import numpy as np
import jax
import jax.numpy as jnp
from jax.experimental import pallas as pl
from jax.experimental.pallas import tpu as pltpu

_K_RATIO = 0.5
_KSIZE = 5
_SIGMA = 1.0


def _gauss_2d_bf16():
    c = _KSIZE // 2
    x, y = np.mgrid[0 - c:_KSIZE - c, 0 - c:_KSIZE - c]
    gk = 1.0 / (2.0 * np.pi * _SIGMA) * np.exp(
        -(np.square(x) + np.square(y)) / (2.0 * np.square(_SIGMA)))
    gk32 = gk.astype(np.float32)
    return gk32.astype(ml_dtypes.bfloat16).astype(np.float32)


def _batch_body(bands_ref, conf_ref, wm_ref, mask_ref, cnt_ref, pad_ref,
                cm_ref):
    L, A = conf_ref.shape[1], conf_ref.shape[2]
    H, W = mask_ref.shape[2], mask_ref.shape[3]
    K = int(H * W * _K_RATIO)
    P = _KSIZE // 2

    m = conf_ref[0, 1:, 0]
    for a in range(1, A):
        m = jnp.maximum(m, conf_ref[0, 1:, a])
    s = jax.nn.sigmoid(m) * wm_ref[0, 1:, 0]

    pad_ref[...] = jnp.zeros_like(pad_ref)
    pad_ref[:, P:P + H, P:P + W] = s.astype(jnp.bfloat16)
    for l in range(L - 1):
        acc = jnp.zeros((H, W), jnp.float32)
        for dy in range(_KSIZE):
            acc += jax.lax.dot_general(
                pad_ref[l, dy:dy + H, :], bands_ref[dy],
                (((1,), (0,)), ((), ())),
                preferred_element_type=jnp.float32)
        cm_ref[l] = acc

    bits = jax.lax.bitcast_convert_type(cm_ref[...], jnp.int32)

    def step(_, lohi):
        lo, hi = lohi
        mid = lo + (hi - lo + 1) // 2
        cnt = jnp.sum((bits >= mid).astype(jnp.int32), axis=(1, 2),
                      keepdims=True)
        big = cnt >= K
        return jnp.where(big, mid, lo), jnp.where(big, hi, mid - 1)

    lo0 = jnp.zeros((L - 1, 1, 1), jnp.int32)
    hi0 = jnp.full((L - 1, 1, 1), 0x3F800000, jnp.int32)
    lo, _ = jax.lax.fori_loop(0, 30, step, (lo0, hi0))

    sel = (bits >= lo).astype(jnp.float32)
    cnt = jnp.sum(sel, axis=(1, 2)).reshape(L - 1, 1)
    cnt_ref[0, 0] = jnp.zeros((cnt_ref.shape[2],), jnp.float32)
    cnt_ref[0, 1:] = jnp.broadcast_to(cnt, (L - 1, cnt_ref.shape[2]))
    mask_ref[0, 0] = jnp.ones((H, W), jnp.float32)
    mask_ref[0, 1:] = sel


def kernel(batch_confidence_maps, B, batch_warp_maks_list, record_len,
           warp_vis_list, warp_conf_list, warp_x_list, gauss_kernel):
    Bs, L, A, H, W = batch_confidence_maps.shape
    P = _KSIZE // 2

    gw = _gauss_2d_bf16()
    bands_np = np.zeros((_KSIZE, W + 2 * P, W), np.float32)
    cols = np.arange(W)
    for dy in range(_KSIZE):
        for dx in range(_KSIZE):
            bands_np[dy, cols + dx, cols] = gw[dy, dx]
    bands = jnp.asarray(bands_np, dtype=jnp.bfloat16)

    masks, counts = pl.pallas_call(
        _batch_body,
        grid=(Bs,),
        in_specs=[
            pl.BlockSpec((_KSIZE, W + 2 * P, W), lambda b: (0, 0, 0)),
            pl.BlockSpec((1, L, A, H, W), lambda b: (b, 0, 0, 0, 0)),
            pl.BlockSpec((1, L, 1, H, W), lambda b: (b, 0, 0, 0, 0)),
        ],
        out_specs=[
            pl.BlockSpec((1, L, H, W), lambda b: (b, 0, 0, 0)),
            pl.BlockSpec((1, L, 128), lambda b: (b, 0, 0)),
        ],
        out_shape=[
            jax.ShapeDtypeStruct((Bs, L, H, W), jnp.float32),
            jax.ShapeDtypeStruct((Bs, L, 128), jnp.float32),
        ],
        scratch_shapes=[
            pltpu.VMEM((L - 1, H + 2 * P, W + 2 * P), jnp.bfloat16),
            pltpu.VMEM((L - 1, H, W), jnp.float32),
        ],
        compiler_params=pltpu.CompilerParams(
            dimension_semantics=("arbitrary",)),
    )(bands, batch_confidence_maps, batch_warp_maks_list)

    masks = masks.reshape(Bs * L, 1, H, W)

    counts = counts[:, :, 0]
    rates = jnp.sum(counts[:, 1:], axis=1) / ((L - 1) * H * W)
    rate = jnp.sum(rates) / Bs
    return masks, rate
